# baseline (speedup 1.0000x reference)
import functools

import jax
import jax.numpy as jnp
from jax import lax
from jax.experimental import pallas as pl
from jax.experimental.pallas import tpu as pltpu
from jax.experimental.pallas import tpu_sc as plsc

_MUL = 34038481
_ADD = 76625530
_SCALE = 1.0 / 147.800537109375
_BIAS = -510.0 / 147.800537109375

_ROWS = 4096
_COLS = 4096
_BLOCK_ROWS = 128


def _decode_kernel(enc_ref, out_ref):
    x = enc_ref[...]
    v = x * jnp.int32(_MUL) + jnp.int32(_ADD)
    t = (v & jnp.int32(0x00FF00FF)) + ((v >> 8) & jnp.int32(0x00FF00FF))
    s = (t + (t >> 16)) & jnp.int32(0x7FF)
    y = s.astype(jnp.float32) * jnp.float32(_SCALE) + jnp.float32(_BIAS)
    out_ref[...] = y.reshape(_BLOCK_ROWS * (_COLS // 128), 128)


_NC = 2
_NW = 32
_WROWS = _ROWS // _NW
_CCHUNK = 2048

_sc_mesh_args = dict(core_axis_name="c", subcore_axis_name="s")


def _sc_body(enc_hbm, lut_hbm, out_hbm, lut_v, idx_v, val_v):
    wid = lax.axis_index("s") * _NC + lax.axis_index("c")
    row0 = wid * _WROWS
    pltpu.sync_copy(lut_hbm, lut_v)

    @pl.loop(0, _WROWS // 8)
    def _band(b):
        r = row0 + b * 8

        for ch in range(_COLS // _CCHUNK):
            pltpu.sync_copy(
                enc_hbm.at[pl.ds(r, 8), pl.ds(ch * _CCHUNK, _CCHUNK)], idx_v
            )
            for s in range(8):
                @pl.loop(0, _CCHUNK, step=16, unroll=8)
                def _g(i):
                    idx = idx_v[s, pl.ds(i, 16)]
                    val_v[s, pl.ds(i, 16)] = plsc.load_gather(lut_v, [idx])

            for s in range(8):
                pltpu.sync_copy(
                    val_v.at[s, :],
                    out_hbm.at[pl.ds((r + s) * _COLS + ch * _CCHUNK, _CCHUNK)],
                )


def _sc_kernel(encoded, lut):
    run = pl.kernel(
        _sc_body,
        out_type=jax.ShapeDtypeStruct((_ROWS * _COLS,), jnp.float32),
        mesh=plsc.VectorSubcoreMesh(**_sc_mesh_args),
        scratch_types=[
            pltpu.VMEM((65536,), jnp.float32),
            pltpu.VMEM((8, _CCHUNK), jnp.int32),
            pltpu.VMEM((8, _CCHUNK), jnp.float32),
        ],
        compiler_params=pltpu.CompilerParams(
            use_tc_tiling_on_sc=True, needs_layout_passes=False
        ),
    )
    out = run(encoded, lut.reshape(65536))
    return out.reshape(_ROWS, _COLS, 1)


def _tc_kernel(encoded, lut):
    del lut
    out = pl.pallas_call(
        _decode_kernel,
        grid=(_ROWS // _BLOCK_ROWS,),
        in_specs=[pl.BlockSpec((_BLOCK_ROWS, _COLS), lambda i: (i, 0))],
        out_specs=pl.BlockSpec(
            (_BLOCK_ROWS * (_COLS // 128), 128), lambda i: (i, 0)
        ),
        out_shape=jax.ShapeDtypeStruct((_ROWS * (_COLS // 128), 128), jnp.float32),
    )(encoded)
    return out.reshape(_ROWS, _COLS, 1)


def kernel(encoded, lut):
    return _sc_kernel(encoded, lut)
